# Initial kernel scaffold; baseline (speedup 1.0000x reference)
import jax
import jax.numpy as jnp
from jax.experimental import pallas as pl


def kernel(n_id, edge_label_index, emb, W1l, b1l, W1r, W2l, b2l, W2r):
    raise NotImplementedError("write your pallas kernel here")


# speedup vs baseline: 1.7311x; 1.7311x over previous
import functools

import jax
import jax.numpy as jnp
from jax import lax
from jax.experimental import pallas as pl
from jax.experimental.pallas import tpu as pltpu
from jax.experimental.pallas import tpu_sc as plsc

N = 10000
E = 320000
D = 128
NC = 2
NS = 16
NW = NC * NS
ROW = 128
NROWS = E // ROW
RPW = 80
NROWS_PAD = RPW * NW
NA = 10240
OWN = NA // NW
ACC = OWN + 8
CAP = 11136
NB = CAP // ROW
SCH = 32
N_PAD = 10240
XB = N_PAD // 128

_f32 = jnp.float32
_i32 = jnp.int32
_mesh = dict(core_axis_name="c", subcore_axis_name="s")
_params = dict(compiler_params=pltpu.CompilerParams(needs_layout_passes=False))


def _k0_body(emb, nidp, x_out, nid_v, rows_v, gsem):
    c = lax.axis_index("c")
    s = lax.axis_index("s")
    w = s * NC + c
    pltpu.sync_copy(nidp, nid_v)

    def xb_body(t, carry):
        b = w + NW * t

        @pl.when(b < XB)
        def _():
            pltpu.async_copy(emb.at[nid_v.at[pl.ds(b * 128, 128)]],
                             rows_v, gsem).wait()
            pltpu.sync_copy(rows_v, x_out.at[pl.ds(b * 128, 128)])

        return carry

    lax.fori_loop(0, -(-XB // NW), xb_body, 0)


def _make_k0():
    return pl.kernel(
        _k0_body,
        out_type=jax.ShapeDtypeStruct((N_PAD, D), _f32),
        mesh=plsc.VectorSubcoreMesh(**_mesh),
        scratch_types=[
            pltpu.VMEM((N_PAD,), _i32),
            pltpu.VMEM((ROW, D), _f32),
            pltpu.SemaphoreType.DMA,
        ],
        **_params,
    )


def _phase_b(table, fill, psrc_f, pdl_f, rows_v, acc, cntacc, gsem):
    ones16 = jnp.ones((16,), _f32)

    def b_body(b, carry):
        @pl.when(b * ROW < fill)
        def _():
            pltpu.async_copy(table.at[psrc_f.at[pl.ds(b * ROW, ROW)]],
                             rows_v, gsem).wait()

            def g_body(g, carry2):
                dl16 = pdl_f[pl.ds(b * ROW + g * 16, 16)]
                for l in range(16):
                    dle = dl16[l]
                    e = g * 16 + l
                    for k in range(8):
                        acc[dle, pl.ds(k * 16, 16)] = (
                            acc[dle, pl.ds(k * 16, 16)]
                            + rows_v[e, pl.ds(k * 16, 16)])
                    if cntacc is not None:
                        cntacc[dle, pl.ds(0, 16)] = (
                            cntacc[dle, pl.ds(0, 16)] + ones16)
                return carry2

            lax.fori_loop(0, 8, g_body, 0)

        return carry

    lax.fori_loop(0, NB, b_body, 0)


def _k1_body(x, srcp, dstp, z128, z16,
             agg_out, cnt_out, psrc_out, pdl_out, fill_out,
             sidx, didx, psrc_f, pdl_f, rows_v,
             acc, cntacc, fbuf, gsem):
    c = lax.axis_index("c")
    s = lax.axis_index("s")
    w = s * NC + c
    lo = w * OWN

    pltpu.sync_copy(z128.at[pl.ds(0, ACC)], acc)
    pltpu.sync_copy(z16.at[pl.ds(0, ACC)], cntacc)

    def chunk_body(q, fill):
        pltpu.sync_copy(srcp.at[pl.ds(q * SCH, SCH)], sidx)
        pltpu.sync_copy(dstp.at[pl.ds(q * SCH, SCH)], didx)

        def row_body(rr, fill2):
            f = fill2
            for j in range(8):
                d = didx[rr, pl.ds(j * 16, 16)]
                sv = sidx[rr, pl.ds(j * 16, 16)]
                m = (d >= lo) & (d < lo + OWN)
                plsc.store_compressed(pdl_f.at[pl.ds(f, 16)], d - lo, mask=m)
                plsc.store_compressed(psrc_f.at[pl.ds(f, 16)], sv, mask=m)
                nv = plsc.all_reduce_population_count(m)
                f = f + nv[0]
            return f

        return lax.fori_loop(0, SCH, row_body, fill)

    fill = lax.fori_loop(0, NROWS_PAD // SCH, chunk_body, 0)

    pad_dl = jnp.full((16,), OWN, _i32)
    pad_src = jnp.zeros((16,), _i32)
    for k in range(8):
        pdl_f[pl.ds(fill + k * 16, 16)] = pad_dl
        psrc_f[pl.ds(fill + k * 16, 16)] = pad_src

    pltpu.sync_copy(psrc_f, psrc_out.at[w])
    pltpu.sync_copy(pdl_f, pdl_out.at[w])
    fbuf[...] = jnp.zeros((16,), _i32) + fill
    pltpu.sync_copy(fbuf, fill_out.at[w])

    _phase_b(x, fill, psrc_f, pdl_f, rows_v, acc, cntacc, gsem)

    pltpu.sync_copy(acc.at[pl.ds(0, OWN)], agg_out.at[pl.ds(lo, OWN)])
    pltpu.sync_copy(cntacc.at[pl.ds(0, OWN)], cnt_out.at[pl.ds(lo, OWN)])


def _make_k1():
    return pl.kernel(
        _k1_body,
        out_type=(
            jax.ShapeDtypeStruct((NA, D), _f32),
            jax.ShapeDtypeStruct((NA, 16), _f32),
            jax.ShapeDtypeStruct((NW, CAP), _i32),
            jax.ShapeDtypeStruct((NW, CAP), _i32),
            jax.ShapeDtypeStruct((NW, 16), _i32),
        ),
        mesh=plsc.VectorSubcoreMesh(**_mesh),
        scratch_types=[
            pltpu.VMEM((SCH, ROW), _i32),
            pltpu.VMEM((SCH, ROW), _i32),
            pltpu.VMEM((CAP,), _i32),
            pltpu.VMEM((CAP,), _i32),
            pltpu.VMEM((ROW, D), _f32),
            pltpu.VMEM((ACC, D), _f32),
            pltpu.VMEM((ACC, 16), _f32),
            pltpu.VMEM((16,), _i32),
            pltpu.SemaphoreType.DMA,
        ],
        **_params,
    )


def _k2_body(h1, psrc_in, pdl_in, fill_in, z128,
             agg_out,
             psrc_f, pdl_f, rows_v, acc, fbuf, gsem):
    c = lax.axis_index("c")
    s = lax.axis_index("s")
    w = s * NC + c
    lo = w * OWN

    pltpu.sync_copy(z128.at[pl.ds(0, ACC)], acc)
    pltpu.sync_copy(psrc_in.at[w], psrc_f)
    pltpu.sync_copy(pdl_in.at[w], pdl_f)
    pltpu.sync_copy(fill_in.at[w], fbuf)
    fill = fbuf[...][0]

    _phase_b(h1, fill, psrc_f, pdl_f, rows_v, acc, None, gsem)

    pltpu.sync_copy(acc.at[pl.ds(0, OWN)], agg_out.at[pl.ds(lo, OWN)])


def _make_k2():
    return pl.kernel(
        _k2_body,
        out_type=jax.ShapeDtypeStruct((NA, D), _f32),
        mesh=plsc.VectorSubcoreMesh(**_mesh),
        scratch_types=[
            pltpu.VMEM((CAP,), _i32),
            pltpu.VMEM((CAP,), _i32),
            pltpu.VMEM((ROW, D), _f32),
            pltpu.VMEM((ACC, D), _f32),
            pltpu.VMEM((16,), _i32),
            pltpu.SemaphoreType.DMA,
        ],
        **_params,
    )


def _k3_body(h, srcp, dstp,
             pred_out,
             sidx, didx, hs_v, hd_v, pred_v, sem1, sem2):
    c = lax.axis_index("c")
    s = lax.axis_index("s")
    w = s * NC + c

    pltpu.sync_copy(srcp.at[pl.ds(w * RPW, RPW)], sidx)
    pltpu.sync_copy(dstp.at[pl.ds(w * RPW, RPW)], didx)
    iota = lax.iota(_i32, 16)
    rows_g = [iota + g * 16 for g in range(8)]

    def row_body(t, carry):
        r = w * RPW + t

        @pl.when(r < NROWS)
        def _():
            ca = pltpu.async_copy(h.at[sidx.at[t]], hs_v, sem1)
            cb = pltpu.async_copy(h.at[didx.at[t]], hd_v, sem2)
            ca.wait()
            cb.wait()

            def kbody(k, accs):
                cols = jnp.zeros((16,), _i32) + k
                new = []
                for g in range(8):
                    a = plsc.load_gather(hs_v, [rows_g[g], cols])
                    b = plsc.load_gather(hd_v, [rows_g[g], cols])
                    new.append(accs[g] + a * b)
                return tuple(new)

            accs = lax.fori_loop(0, D, kbody,
                                 tuple(jnp.zeros((16,), _f32) for _ in range(8)))
            for g in range(8):
                pred_v[pl.ds(g * 16, 16)] = accs[g]
            pltpu.sync_copy(pred_v, pred_out.at[pl.ds(r * ROW, ROW)])

        return carry

    lax.fori_loop(0, RPW, row_body, 0)


def _make_k3():
    return pl.kernel(
        _k3_body,
        out_type=jax.ShapeDtypeStruct((E,), _f32),
        mesh=plsc.VectorSubcoreMesh(**_mesh),
        scratch_types=[
            pltpu.VMEM((RPW, ROW), _i32),
            pltpu.VMEM((RPW, ROW), _i32),
            pltpu.VMEM((ROW, D), _f32),
            pltpu.VMEM((ROW, D), _f32),
            pltpu.VMEM((ROW,), _f32),
            pltpu.SemaphoreType.DMA,
            pltpu.SemaphoreType.DMA,
        ],
        **_params,
    )


def _sage_tc_body(relu, agg, cnt, xin, wl, b, wr, o):
    c1 = jnp.maximum(cnt[:, 0:1], 1.0)
    mean = agg[...] / c1
    h = (jnp.dot(mean, wl[...], preferred_element_type=_f32,
                 precision=lax.Precision.HIGHEST)
         + b[...]
         + jnp.dot(xin[...], wr[...], preferred_element_type=_f32,
                   precision=lax.Precision.HIGHEST))
    o[...] = jnp.maximum(h, 0.0) if relu else h


def _sage_tc(relu, agg, cnt, xin, wlT, b, wrT):
    blk = 1000
    return pl.pallas_call(
        functools.partial(_sage_tc_body, relu),
        grid=(N // blk,),
        in_specs=[
            pl.BlockSpec((blk, D), lambda i: (i, 0)),
            pl.BlockSpec((blk, 16), lambda i: (i, 0)),
            pl.BlockSpec((blk, D), lambda i: (i, 0)),
            pl.BlockSpec((D, D), lambda i: (0, 0)),
            pl.BlockSpec((1, D), lambda i: (0, 0)),
            pl.BlockSpec((D, D), lambda i: (0, 0)),
        ],
        out_specs=pl.BlockSpec((blk, D), lambda i: (i, 0)),
        out_shape=jax.ShapeDtypeStruct((N, D), _f32),
    )(agg, cnt, xin, wlT, b, wrT)


def kernel(n_id, edge_label_index, emb, W1l, b1l, W1r, W2l, b2l, W2r):
    pad_rows = NROWS_PAD - NROWS
    src2 = jnp.concatenate(
        [edge_label_index[0], jnp.zeros((pad_rows * ROW,), _i32)]).reshape(
        NROWS_PAD, ROW)
    dst2 = jnp.concatenate(
        [edge_label_index[-1],
         jnp.full((pad_rows * ROW,), NA, _i32)]).reshape(NROWS_PAD, ROW)
    nidp = jnp.concatenate([n_id, jnp.zeros((N_PAD - N,), _i32)])
    z128 = jnp.zeros((NA, D), _f32)
    z16 = jnp.zeros((NA, 16), _f32)
    b1 = b1l.reshape(1, D)
    b2 = b2l.reshape(1, D)

    x_pad = _make_k0()(emb, nidp)
    agg1, cnt, psrc, pdl, fills = _make_k1()(
        x_pad, src2, dst2, z128, z16)
    h1 = _sage_tc(True, agg1[:N], cnt[:N], x_pad[:N], W1l.T, b1, W1r.T)
    agg2 = _make_k2()(h1, psrc, pdl, fills, z128)
    h = _sage_tc(False, agg2[:N], cnt[:N], h1, W2l.T, b2, W2r.T)
    pred = _make_k3()(h, src2, dst2)
    return pred


# speedup vs baseline: 2.8213x; 1.6297x over previous
import functools

import jax
import jax.numpy as jnp
from jax import lax
from jax.experimental import pallas as pl
from jax.experimental.pallas import tpu as pltpu
from jax.experimental.pallas import tpu_sc as plsc

N = 10000
E = 320000
D = 128
NC = 2
NS = 16
NW = NC * NS
ROW = 128
NROWS = E // ROW
RPW = 80
NROWS_PAD = RPW * NW
NA = 10240
OWN = NA // NW
ACC = OWN + 8
CAP = 11136
NB = CAP // ROW
SCH = 32
N_PAD = 10240
XB = N_PAD // 128

_f32 = jnp.float32
_i32 = jnp.int32
_mesh = dict(core_axis_name="c", subcore_axis_name="s")
_params = dict(compiler_params=pltpu.CompilerParams(needs_layout_passes=False))


def _k0_body(emb, nidp, x_out, nid_v, rows_v, gsem):
    c = lax.axis_index("c")
    s = lax.axis_index("s")
    w = s * NC + c
    pltpu.sync_copy(nidp, nid_v)

    def xb_body(t, carry):
        b = w + NW * t

        @pl.when(b < XB)
        def _():
            pltpu.async_copy(emb.at[nid_v.at[pl.ds(b * 128, 128)]],
                             rows_v, gsem).wait()
            pltpu.sync_copy(rows_v, x_out.at[pl.ds(b * 128, 128)])

        return carry

    lax.fori_loop(0, -(-XB // NW), xb_body, 0)


def _make_k0():
    return pl.kernel(
        _k0_body,
        out_type=jax.ShapeDtypeStruct((N_PAD, D), _f32),
        mesh=plsc.VectorSubcoreMesh(**_mesh),
        scratch_types=[
            pltpu.VMEM((N_PAD,), _i32),
            pltpu.VMEM((ROW, D), _f32),
            pltpu.SemaphoreType.DMA,
        ],
        **_params,
    )


def _phase_b(table, fill, psrc_f, pdl_f, rows_v, acc, cntacc, gsem):
    ones16 = jnp.ones((16,), _f32)

    def b_body(b, carry):
        @pl.when(b * ROW < fill)
        def _():
            pltpu.async_copy(table.at[psrc_f.at[pl.ds(b * ROW, ROW)]],
                             rows_v, gsem).wait()

            def g_body(g, carry2):
                dl16 = pdl_f[pl.ds(b * ROW + g * 16, 16)]
                for l in range(16):
                    dle = dl16[l]
                    e = g * 16 + l
                    for k in range(8):
                        acc[dle, pl.ds(k * 16, 16)] = (
                            acc[dle, pl.ds(k * 16, 16)]
                            + rows_v[e, pl.ds(k * 16, 16)])
                    if cntacc is not None:
                        cntacc[dle, pl.ds(0, 16)] = (
                            cntacc[dle, pl.ds(0, 16)] + ones16)
                return carry2

            lax.fori_loop(0, 8, g_body, 0)

        return carry

    lax.fori_loop(0, NB, b_body, 0)


def _k1_body(x, srcp, dstp, z128, z16,
             agg_out, cnt_out, psrc_out, pdl_out, fill_out,
             sidx, didx, psrc_f, pdl_f, rows_v,
             acc, cntacc, fbuf, gsem):
    c = lax.axis_index("c")
    s = lax.axis_index("s")
    w = s * NC + c
    lo = w * OWN

    pltpu.sync_copy(z128.at[pl.ds(0, ACC)], acc)
    pltpu.sync_copy(z16.at[pl.ds(0, ACC)], cntacc)

    def chunk_body(q, fill):
        pltpu.sync_copy(srcp.at[pl.ds(q * SCH, SCH)], sidx)
        pltpu.sync_copy(dstp.at[pl.ds(q * SCH, SCH)], didx)

        def row_body(rr, fill2):
            f = fill2
            for j in range(8):
                d = didx[rr, pl.ds(j * 16, 16)]
                sv = sidx[rr, pl.ds(j * 16, 16)]
                m = (d >= lo) & (d < lo + OWN)
                plsc.store_compressed(pdl_f.at[pl.ds(f, 16)], d - lo, mask=m)
                plsc.store_compressed(psrc_f.at[pl.ds(f, 16)], sv, mask=m)
                nv = plsc.all_reduce_population_count(m)
                f = f + nv[0]
            return f

        return lax.fori_loop(0, SCH, row_body, fill)

    fill = lax.fori_loop(0, NROWS_PAD // SCH, chunk_body, 0)

    pad_dl = jnp.full((16,), OWN, _i32)
    pad_src = jnp.zeros((16,), _i32)
    for k in range(8):
        pdl_f[pl.ds(fill + k * 16, 16)] = pad_dl
        psrc_f[pl.ds(fill + k * 16, 16)] = pad_src

    pltpu.sync_copy(psrc_f, psrc_out.at[w])
    pltpu.sync_copy(pdl_f, pdl_out.at[w])
    fbuf[...] = jnp.zeros((16,), _i32) + fill
    pltpu.sync_copy(fbuf, fill_out.at[w])

    _phase_b(x, fill, psrc_f, pdl_f, rows_v, acc, cntacc, gsem)

    pltpu.sync_copy(acc.at[pl.ds(0, OWN)], agg_out.at[pl.ds(lo, OWN)])
    pltpu.sync_copy(cntacc.at[pl.ds(0, OWN)], cnt_out.at[pl.ds(lo, OWN)])


def _make_k1():
    return pl.kernel(
        _k1_body,
        out_type=(
            jax.ShapeDtypeStruct((NA, D), _f32),
            jax.ShapeDtypeStruct((NA, 16), _f32),
            jax.ShapeDtypeStruct((NW, CAP), _i32),
            jax.ShapeDtypeStruct((NW, CAP), _i32),
            jax.ShapeDtypeStruct((NW, 16), _i32),
        ),
        mesh=plsc.VectorSubcoreMesh(**_mesh),
        scratch_types=[
            pltpu.VMEM((SCH, ROW), _i32),
            pltpu.VMEM((SCH, ROW), _i32),
            pltpu.VMEM((CAP,), _i32),
            pltpu.VMEM((CAP,), _i32),
            pltpu.VMEM((ROW, D), _f32),
            pltpu.VMEM((ACC, D), _f32),
            pltpu.VMEM((ACC, 16), _f32),
            pltpu.VMEM((16,), _i32),
            pltpu.SemaphoreType.DMA,
        ],
        **_params,
    )


def _k2_body(h1, psrc_in, pdl_in, fill_in, z128,
             agg_out,
             psrc_f, pdl_f, rows_v, acc, fbuf, gsem):
    c = lax.axis_index("c")
    s = lax.axis_index("s")
    w = s * NC + c
    lo = w * OWN

    pltpu.sync_copy(z128.at[pl.ds(0, ACC)], acc)
    pltpu.sync_copy(psrc_in.at[w], psrc_f)
    pltpu.sync_copy(pdl_in.at[w], pdl_f)
    pltpu.sync_copy(fill_in.at[w], fbuf)
    fill = fbuf[...][0]

    _phase_b(h1, fill, psrc_f, pdl_f, rows_v, acc, None, gsem)

    pltpu.sync_copy(acc.at[pl.ds(0, OWN)], agg_out.at[pl.ds(lo, OWN)])


def _make_k2():
    return pl.kernel(
        _k2_body,
        out_type=jax.ShapeDtypeStruct((NA, D), _f32),
        mesh=plsc.VectorSubcoreMesh(**_mesh),
        scratch_types=[
            pltpu.VMEM((CAP,), _i32),
            pltpu.VMEM((CAP,), _i32),
            pltpu.VMEM((ROW, D), _f32),
            pltpu.VMEM((ACC, D), _f32),
            pltpu.VMEM((16,), _i32),
            pltpu.SemaphoreType.DMA,
        ],
        **_params,
    )


def _k3_body(h, srcp, dstp,
             hs_out, hd_out,
             sidx, didx, hs_v, hd_v, sem1, sem2):
    c = lax.axis_index("c")
    s = lax.axis_index("s")
    w = s * NC + c

    pltpu.sync_copy(srcp.at[pl.ds(w * RPW, RPW)], sidx)
    pltpu.sync_copy(dstp.at[pl.ds(w * RPW, RPW)], didx)

    def row_body(t, carry):
        r = w * RPW + t

        @pl.when(r < NROWS)
        def _():
            ca = pltpu.async_copy(h.at[sidx.at[t]], hs_v, sem1)
            cb = pltpu.async_copy(h.at[didx.at[t]], hd_v, sem2)
            ca.wait()
            pltpu.sync_copy(hs_v, hs_out.at[pl.ds(r * ROW, ROW)])
            cb.wait()
            pltpu.sync_copy(hd_v, hd_out.at[pl.ds(r * ROW, ROW)])

        return carry

    lax.fori_loop(0, RPW, row_body, 0)


def _make_k3():
    return pl.kernel(
        _k3_body,
        out_type=(
            jax.ShapeDtypeStruct((NROWS_PAD * ROW, D), _f32),
            jax.ShapeDtypeStruct((NROWS_PAD * ROW, D), _f32),
        ),
        mesh=plsc.VectorSubcoreMesh(**_mesh),
        scratch_types=[
            pltpu.VMEM((RPW, ROW), _i32),
            pltpu.VMEM((RPW, ROW), _i32),
            pltpu.VMEM((ROW, D), _f32),
            pltpu.VMEM((ROW, D), _f32),
            pltpu.SemaphoreType.DMA,
            pltpu.SemaphoreType.DMA,
        ],
        **_params,
    )


def _dot_tc_body(hs, hd, o):
    o[...] = jnp.sum(hs[...] * hd[...], axis=-1)


def _dot_tc(hs3, hd3):
    blk = 128
    return pl.pallas_call(
        _dot_tc_body,
        grid=(NROWS_PAD // blk,),
        in_specs=[
            pl.BlockSpec((blk, ROW, D), lambda i: (i, 0, 0)),
            pl.BlockSpec((blk, ROW, D), lambda i: (i, 0, 0)),
        ],
        out_specs=pl.BlockSpec((blk, ROW), lambda i: (i, 0)),
        out_shape=jax.ShapeDtypeStruct((NROWS_PAD, ROW), _f32),
    )(hs3, hd3)


def _sage_tc_body(relu, agg, cnt, xin, wl, b, wr, o):
    c1 = jnp.maximum(cnt[:, 0:1], 1.0)
    mean = agg[...] / c1
    h = (jnp.dot(mean, wl[...], preferred_element_type=_f32,
                 precision=lax.Precision.HIGHEST)
         + b[...]
         + jnp.dot(xin[...], wr[...], preferred_element_type=_f32,
                   precision=lax.Precision.HIGHEST))
    o[...] = jnp.maximum(h, 0.0) if relu else h


def _sage_tc(relu, agg, cnt, xin, wlT, b, wrT):
    blk = 1000
    return pl.pallas_call(
        functools.partial(_sage_tc_body, relu),
        grid=(N // blk,),
        in_specs=[
            pl.BlockSpec((blk, D), lambda i: (i, 0)),
            pl.BlockSpec((blk, 16), lambda i: (i, 0)),
            pl.BlockSpec((blk, D), lambda i: (i, 0)),
            pl.BlockSpec((D, D), lambda i: (0, 0)),
            pl.BlockSpec((1, D), lambda i: (0, 0)),
            pl.BlockSpec((D, D), lambda i: (0, 0)),
        ],
        out_specs=pl.BlockSpec((blk, D), lambda i: (i, 0)),
        out_shape=jax.ShapeDtypeStruct((N, D), _f32),
    )(agg, cnt, xin, wlT, b, wrT)


def kernel(n_id, edge_label_index, emb, W1l, b1l, W1r, W2l, b2l, W2r):
    pad_rows = NROWS_PAD - NROWS
    src2 = jnp.concatenate(
        [edge_label_index[0], jnp.zeros((pad_rows * ROW,), _i32)]).reshape(
        NROWS_PAD, ROW)
    dst2 = jnp.concatenate(
        [edge_label_index[-1],
         jnp.full((pad_rows * ROW,), NA, _i32)]).reshape(NROWS_PAD, ROW)
    nidp = jnp.concatenate([n_id, jnp.zeros((N_PAD - N,), _i32)])
    z128 = jnp.zeros((NA, D), _f32)
    z16 = jnp.zeros((NA, 16), _f32)
    b1 = b1l.reshape(1, D)
    b2 = b2l.reshape(1, D)

    x_pad = _make_k0()(emb, nidp)
    agg1, cnt, psrc, pdl, fills = _make_k1()(
        x_pad, src2, dst2, z128, z16)
    h1 = _sage_tc(True, agg1[:N], cnt[:N], x_pad[:N], W1l.T, b1, W1r.T)
    agg2 = _make_k2()(h1, psrc, pdl, fills, z128)
    h = _sage_tc(False, agg2[:N], cnt[:N], h1, W2l.T, b2, W2r.T)
    hs, hd = _make_k3()(h, src2, dst2)
    pred = _dot_tc(hs.reshape(NROWS_PAD, ROW, D), hd.reshape(NROWS_PAD, ROW, D))
    return pred.reshape(NROWS_PAD * ROW)[:E]


# speedup vs baseline: 3.1500x; 1.1165x over previous
import functools

import jax
import jax.numpy as jnp
from jax import lax
from jax.experimental import pallas as pl
from jax.experimental.pallas import tpu as pltpu
from jax.experimental.pallas import tpu_sc as plsc

N = 10000
E = 320000
D = 128
NC = 2
NS = 16
NW = NC * NS
ROW = 128
NROWS = E // ROW
RPW = 80
NROWS_PAD = RPW * NW
NA = 10240
OWN = NA // NW
ACC = OWN + 8
CAP = 11136
NB = CAP // ROW
SCH = 32
N_PAD = 10240
XB = N_PAD // 128

_f32 = jnp.float32
_i32 = jnp.int32
_mesh = dict(core_axis_name="c", subcore_axis_name="s")
_params = dict(compiler_params=pltpu.CompilerParams(needs_layout_passes=False))


def _k0_body(emb, nidp, x_out, nid_v, rows_v, gsem):
    c = lax.axis_index("c")
    s = lax.axis_index("s")
    w = s * NC + c
    pltpu.sync_copy(nidp, nid_v)

    def xb_body(t, carry):
        b = w + NW * t

        @pl.when(b < XB)
        def _():
            pltpu.async_copy(emb.at[nid_v.at[pl.ds(b * 128, 128)]],
                             rows_v, gsem).wait()
            pltpu.sync_copy(rows_v, x_out.at[pl.ds(b * 128, 128)])

        return carry

    lax.fori_loop(0, -(-XB // NW), xb_body, 0)


def _make_k0():
    return pl.kernel(
        _k0_body,
        out_type=jax.ShapeDtypeStruct((N_PAD, D), _f32),
        mesh=plsc.VectorSubcoreMesh(**_mesh),
        scratch_types=[
            pltpu.VMEM((N_PAD,), _i32),
            pltpu.VMEM((ROW, D), _f32),
            pltpu.SemaphoreType.DMA,
        ],
        **_params,
    )


def _phase_b(table, fill, psrc_f, pdl_f, rows_v, acc, cntacc, gsem):
    ones16 = jnp.ones((16,), _f32)

    def b_body(b, carry):
        @pl.when(b * ROW < fill)
        def _():
            pltpu.async_copy(table.at[psrc_f.at[pl.ds(b * ROW, ROW)]],
                             rows_v, gsem).wait()

            def g_body(g, carry2):
                dl16 = pdl_f[pl.ds(b * ROW + g * 16, 16)]
                for l in range(16):
                    dle = dl16[l]
                    e = g * 16 + l
                    for k in range(8):
                        plsc.addupdate(acc.at[dle, pl.ds(k * 16, 16)],
                                       rows_v[e, pl.ds(k * 16, 16)])
                    if cntacc is not None:
                        plsc.addupdate(cntacc.at[dle, pl.ds(0, 16)], ones16)
                return carry2

            lax.fori_loop(0, 8, g_body, 0)

        return carry

    lax.fori_loop(0, NB, b_body, 0)


def _k1_body(x, srcp, dstp, z128, z16,
             agg_out, cnt_out, psrc_out, pdl_out, fill_out,
             sidx, didx, psrc_f, pdl_f, rows_v,
             acc, cntacc, fbuf, gsem):
    c = lax.axis_index("c")
    s = lax.axis_index("s")
    w = s * NC + c
    lo = w * OWN

    pltpu.sync_copy(z128.at[pl.ds(0, ACC)], acc)
    pltpu.sync_copy(z16.at[pl.ds(0, ACC)], cntacc)

    def chunk_body(q, fill):
        pltpu.sync_copy(srcp.at[pl.ds(q * SCH, SCH)], sidx)
        pltpu.sync_copy(dstp.at[pl.ds(q * SCH, SCH)], didx)

        def row_body(rr, fill2):
            f = fill2
            for j in range(8):
                d = didx[rr, pl.ds(j * 16, 16)]
                sv = sidx[rr, pl.ds(j * 16, 16)]
                m = (d >= lo) & (d < lo + OWN)
                plsc.store_compressed(pdl_f.at[pl.ds(f, 16)], d - lo, mask=m)
                plsc.store_compressed(psrc_f.at[pl.ds(f, 16)], sv, mask=m)
                nv = plsc.all_reduce_population_count(m)
                f = f + nv[0]
            return f

        return lax.fori_loop(0, SCH, row_body, fill)

    fill = lax.fori_loop(0, NROWS_PAD // SCH, chunk_body, 0)

    pad_dl = jnp.full((16,), OWN, _i32)
    pad_src = jnp.zeros((16,), _i32)
    for k in range(8):
        pdl_f[pl.ds(fill + k * 16, 16)] = pad_dl
        psrc_f[pl.ds(fill + k * 16, 16)] = pad_src

    pltpu.sync_copy(psrc_f, psrc_out.at[w])
    pltpu.sync_copy(pdl_f, pdl_out.at[w])
    fbuf[...] = jnp.zeros((16,), _i32) + fill
    pltpu.sync_copy(fbuf, fill_out.at[w])

    _phase_b(x, fill, psrc_f, pdl_f, rows_v, acc, cntacc, gsem)

    pltpu.sync_copy(acc.at[pl.ds(0, OWN)], agg_out.at[pl.ds(lo, OWN)])
    pltpu.sync_copy(cntacc.at[pl.ds(0, OWN)], cnt_out.at[pl.ds(lo, OWN)])


def _make_k1():
    return pl.kernel(
        _k1_body,
        out_type=(
            jax.ShapeDtypeStruct((NA, D), _f32),
            jax.ShapeDtypeStruct((NA, 16), _f32),
            jax.ShapeDtypeStruct((NW, CAP), _i32),
            jax.ShapeDtypeStruct((NW, CAP), _i32),
            jax.ShapeDtypeStruct((NW, 16), _i32),
        ),
        mesh=plsc.VectorSubcoreMesh(**_mesh),
        scratch_types=[
            pltpu.VMEM((SCH, ROW), _i32),
            pltpu.VMEM((SCH, ROW), _i32),
            pltpu.VMEM((CAP,), _i32),
            pltpu.VMEM((CAP,), _i32),
            pltpu.VMEM((ROW, D), _f32),
            pltpu.VMEM((ACC, D), _f32),
            pltpu.VMEM((ACC, 16), _f32),
            pltpu.VMEM((16,), _i32),
            pltpu.SemaphoreType.DMA,
        ],
        **_params,
    )


def _k2_body(h1, psrc_in, pdl_in, fill_in, z128,
             agg_out,
             psrc_f, pdl_f, rows_v, acc, fbuf, gsem):
    c = lax.axis_index("c")
    s = lax.axis_index("s")
    w = s * NC + c
    lo = w * OWN

    pltpu.sync_copy(z128.at[pl.ds(0, ACC)], acc)
    pltpu.sync_copy(psrc_in.at[w], psrc_f)
    pltpu.sync_copy(pdl_in.at[w], pdl_f)
    pltpu.sync_copy(fill_in.at[w], fbuf)
    fill = fbuf[...][0]

    _phase_b(h1, fill, psrc_f, pdl_f, rows_v, acc, None, gsem)

    pltpu.sync_copy(acc.at[pl.ds(0, OWN)], agg_out.at[pl.ds(lo, OWN)])


def _make_k2():
    return pl.kernel(
        _k2_body,
        out_type=jax.ShapeDtypeStruct((NA, D), _f32),
        mesh=plsc.VectorSubcoreMesh(**_mesh),
        scratch_types=[
            pltpu.VMEM((CAP,), _i32),
            pltpu.VMEM((CAP,), _i32),
            pltpu.VMEM((ROW, D), _f32),
            pltpu.VMEM((ACC, D), _f32),
            pltpu.VMEM((16,), _i32),
            pltpu.SemaphoreType.DMA,
        ],
        **_params,
    )


def _k3_body(h, srcp, dstp,
             hs_out, hd_out,
             sidx, didx, hs_v, hd_v, sem1, sem2):
    c = lax.axis_index("c")
    s = lax.axis_index("s")
    w = s * NC + c

    pltpu.sync_copy(srcp.at[pl.ds(w * RPW, RPW)], sidx)
    pltpu.sync_copy(dstp.at[pl.ds(w * RPW, RPW)], didx)

    def row_body(t, carry):
        r = w * RPW + t

        @pl.when(r < NROWS)
        def _():
            ca = pltpu.async_copy(h.at[sidx.at[t]], hs_v, sem1)
            cb = pltpu.async_copy(h.at[didx.at[t]], hd_v, sem2)
            ca.wait()
            pltpu.sync_copy(hs_v, hs_out.at[pl.ds(r * ROW, ROW)])
            cb.wait()
            pltpu.sync_copy(hd_v, hd_out.at[pl.ds(r * ROW, ROW)])

        return carry

    lax.fori_loop(0, RPW, row_body, 0)


def _make_k3():
    return pl.kernel(
        _k3_body,
        out_type=(
            jax.ShapeDtypeStruct((NROWS_PAD * ROW, D), _f32),
            jax.ShapeDtypeStruct((NROWS_PAD * ROW, D), _f32),
        ),
        mesh=plsc.VectorSubcoreMesh(**_mesh),
        scratch_types=[
            pltpu.VMEM((RPW, ROW), _i32),
            pltpu.VMEM((RPW, ROW), _i32),
            pltpu.VMEM((ROW, D), _f32),
            pltpu.VMEM((ROW, D), _f32),
            pltpu.SemaphoreType.DMA,
            pltpu.SemaphoreType.DMA,
        ],
        **_params,
    )


def _dot_tc_body(hs, hd, o):
    o[...] = jnp.sum(hs[...] * hd[...], axis=-1)


def _dot_tc(hs3, hd3):
    blk = 128
    return pl.pallas_call(
        _dot_tc_body,
        grid=(NROWS_PAD // blk,),
        in_specs=[
            pl.BlockSpec((blk, ROW, D), lambda i: (i, 0, 0)),
            pl.BlockSpec((blk, ROW, D), lambda i: (i, 0, 0)),
        ],
        out_specs=pl.BlockSpec((blk, ROW), lambda i: (i, 0)),
        out_shape=jax.ShapeDtypeStruct((NROWS_PAD, ROW), _f32),
    )(hs3, hd3)


def _sage_tc_body(relu, agg, cnt, xin, wl, b, wr, o):
    c1 = jnp.maximum(cnt[:, 0:1], 1.0)
    mean = agg[...] / c1
    h = (jnp.dot(mean, wl[...], preferred_element_type=_f32,
                 precision=lax.Precision.HIGHEST)
         + b[...]
         + jnp.dot(xin[...], wr[...], preferred_element_type=_f32,
                   precision=lax.Precision.HIGHEST))
    o[...] = jnp.maximum(h, 0.0) if relu else h


def _sage_tc(relu, agg, cnt, xin, wlT, b, wrT):
    blk = 1000
    return pl.pallas_call(
        functools.partial(_sage_tc_body, relu),
        grid=(N // blk,),
        in_specs=[
            pl.BlockSpec((blk, D), lambda i: (i, 0)),
            pl.BlockSpec((blk, 16), lambda i: (i, 0)),
            pl.BlockSpec((blk, D), lambda i: (i, 0)),
            pl.BlockSpec((D, D), lambda i: (0, 0)),
            pl.BlockSpec((1, D), lambda i: (0, 0)),
            pl.BlockSpec((D, D), lambda i: (0, 0)),
        ],
        out_specs=pl.BlockSpec((blk, D), lambda i: (i, 0)),
        out_shape=jax.ShapeDtypeStruct((N, D), _f32),
    )(agg, cnt, xin, wlT, b, wrT)


def kernel(n_id, edge_label_index, emb, W1l, b1l, W1r, W2l, b2l, W2r):
    pad_rows = NROWS_PAD - NROWS
    src2 = jnp.concatenate(
        [edge_label_index[0], jnp.zeros((pad_rows * ROW,), _i32)]).reshape(
        NROWS_PAD, ROW)
    dst2 = jnp.concatenate(
        [edge_label_index[-1],
         jnp.full((pad_rows * ROW,), NA, _i32)]).reshape(NROWS_PAD, ROW)
    nidp = jnp.concatenate([n_id, jnp.zeros((N_PAD - N,), _i32)])
    z128 = jnp.zeros((NA, D), _f32)
    z16 = jnp.zeros((NA, 16), _f32)
    b1 = b1l.reshape(1, D)
    b2 = b2l.reshape(1, D)

    x_pad = _make_k0()(emb, nidp)
    agg1, cnt, psrc, pdl, fills = _make_k1()(
        x_pad, src2, dst2, z128, z16)
    h1 = _sage_tc(True, agg1[:N], cnt[:N], x_pad[:N], W1l.T, b1, W1r.T)
    agg2 = _make_k2()(h1, psrc, pdl, fills, z128)
    h = _sage_tc(False, agg2[:N], cnt[:N], h1, W2l.T, b2, W2r.T)
    hs, hd = _make_k3()(h, src2, dst2)
    pred = _dot_tc(hs.reshape(NROWS_PAD, ROW, D), hd.reshape(NROWS_PAD, ROW, D))
    return pred.reshape(NROWS_PAD * ROW)[:E]


# speedup vs baseline: 3.2340x; 1.0266x over previous
import functools

import jax
import jax.numpy as jnp
from jax import lax
from jax.experimental import pallas as pl
from jax.experimental.pallas import tpu as pltpu
from jax.experimental.pallas import tpu_sc as plsc

N = 10000
E = 320000
D = 128
NC = 2
NS = 16
NW = NC * NS
ROW = 128
NROWS = E // ROW
RPW = 80
NROWS_PAD = RPW * NW
NA = 10240
OWN = NA // NW
ACC = OWN + 8
CAP = 11136
NB = CAP // ROW
SCH = 32
N_PAD = 10240
XB = N_PAD // 128

_f32 = jnp.float32
_i32 = jnp.int32
_mesh = dict(core_axis_name="c", subcore_axis_name="s")
_params = dict(compiler_params=pltpu.CompilerParams(needs_layout_passes=False))


def _k0_body(emb, nidp, x_out, nid_v, rows_v, gsem):
    c = lax.axis_index("c")
    s = lax.axis_index("s")
    w = s * NC + c
    pltpu.sync_copy(nidp, nid_v)

    def xb_body(t, carry):
        b = w + NW * t

        @pl.when(b < XB)
        def _():
            pltpu.async_copy(emb.at[nid_v.at[pl.ds(b * 128, 128)]],
                             rows_v, gsem).wait()
            pltpu.sync_copy(rows_v, x_out.at[pl.ds(b * 128, 128)])

        return carry

    lax.fori_loop(0, -(-XB // NW), xb_body, 0)


def _make_k0():
    return pl.kernel(
        _k0_body,
        out_type=jax.ShapeDtypeStruct((N_PAD, D), _f32),
        mesh=plsc.VectorSubcoreMesh(**_mesh),
        scratch_types=[
            pltpu.VMEM((N_PAD,), _i32),
            pltpu.VMEM((ROW, D), _f32),
            pltpu.SemaphoreType.DMA,
        ],
        **_params,
    )


def _phase_b(table, fill, psrc_f, pdl_f, rows_v, acc, cntacc, gsem):
    ones16 = jnp.ones((16,), _f32)

    def b_body(b, carry):
        @pl.when(b * ROW < fill)
        def _():
            pltpu.async_copy(table.at[psrc_f.at[pl.ds(b * ROW, ROW)]],
                             rows_v, gsem).wait()

            def g_body(g, carry2):
                dl16 = pdl_f[pl.ds(b * ROW + g * 16, 16)]
                for l in range(16):
                    dle = dl16[l]
                    e = g * 16 + l
                    for k in range(8):
                        plsc.addupdate(acc.at[dle, pl.ds(k * 16, 16)],
                                       rows_v[e, pl.ds(k * 16, 16)])
                    if cntacc is not None:
                        plsc.addupdate(cntacc.at[dle, pl.ds(0, 16)], ones16)
                return carry2

            lax.fori_loop(0, 8, g_body, 0)

        return carry

    lax.fori_loop(0, NB, b_body, 0)


def _k1_body(x, srcp, dstp, z128, z16,
             agg_out, cnt_out, psrc_out, pdl_out, fill_out,
             sidx, didx, psrc_f, pdl_f, rows_v,
             acc, cntacc, fbuf, gsem):
    c = lax.axis_index("c")
    s = lax.axis_index("s")
    w = s * NC + c
    lo = w * OWN

    pltpu.sync_copy(z128.at[pl.ds(0, ACC)], acc)
    pltpu.sync_copy(z16.at[pl.ds(0, ACC)], cntacc)

    def chunk_body(q, fill):
        pltpu.sync_copy(srcp.at[pl.ds(q * SCH, SCH)], sidx)
        pltpu.sync_copy(dstp.at[pl.ds(q * SCH, SCH)], didx)

        def row_body(rr, fill2):
            f = fill2
            for j in range(8):
                d = didx[rr, pl.ds(j * 16, 16)]
                sv = sidx[rr, pl.ds(j * 16, 16)]
                m = (d >= lo) & (d < lo + OWN)
                plsc.store_compressed(pdl_f.at[pl.ds(f, 16)], d - lo, mask=m)
                plsc.store_compressed(psrc_f.at[pl.ds(f, 16)], sv, mask=m)
                nv = plsc.all_reduce_population_count(m)
                f = f + nv[0]
            return f

        return lax.fori_loop(0, SCH, row_body, fill)

    fill = lax.fori_loop(0, NROWS_PAD // SCH, chunk_body, 0)

    pad_dl = jnp.full((16,), OWN, _i32)
    pad_src = jnp.zeros((16,), _i32)
    for k in range(8):
        pdl_f[pl.ds(fill + k * 16, 16)] = pad_dl
        psrc_f[pl.ds(fill + k * 16, 16)] = pad_src

    pltpu.sync_copy(psrc_f, psrc_out.at[w])
    pltpu.sync_copy(pdl_f, pdl_out.at[w])
    fbuf[...] = jnp.zeros((16,), _i32) + fill
    pltpu.sync_copy(fbuf, fill_out.at[w])

    _phase_b(x, fill, psrc_f, pdl_f, rows_v, acc, cntacc, gsem)

    pltpu.sync_copy(acc.at[pl.ds(0, OWN)], agg_out.at[pl.ds(lo, OWN)])
    pltpu.sync_copy(cntacc.at[pl.ds(0, OWN)], cnt_out.at[pl.ds(lo, OWN)])


def _make_k1():
    return pl.kernel(
        _k1_body,
        out_type=(
            jax.ShapeDtypeStruct((NA, D), _f32),
            jax.ShapeDtypeStruct((NA, 16), _f32),
            jax.ShapeDtypeStruct((NW, CAP), _i32),
            jax.ShapeDtypeStruct((NW, CAP), _i32),
            jax.ShapeDtypeStruct((NW, 16), _i32),
        ),
        mesh=plsc.VectorSubcoreMesh(**_mesh),
        scratch_types=[
            pltpu.VMEM((SCH, ROW), _i32),
            pltpu.VMEM((SCH, ROW), _i32),
            pltpu.VMEM((CAP,), _i32),
            pltpu.VMEM((CAP,), _i32),
            pltpu.VMEM((ROW, D), _f32),
            pltpu.VMEM((ACC, D), _f32),
            pltpu.VMEM((ACC, 16), _f32),
            pltpu.VMEM((16,), _i32),
            pltpu.SemaphoreType.DMA,
        ],
        **_params,
    )


def _k2_body(h1, psrc_in, pdl_in, fill_in, z128,
             agg_out,
             psrc_f, pdl_f, rows_v, acc, fbuf, gsem):
    c = lax.axis_index("c")
    s = lax.axis_index("s")
    w = s * NC + c
    lo = w * OWN

    pltpu.sync_copy(z128.at[pl.ds(0, ACC)], acc)
    pltpu.sync_copy(psrc_in.at[w], psrc_f)
    pltpu.sync_copy(pdl_in.at[w], pdl_f)
    pltpu.sync_copy(fill_in.at[w], fbuf)
    fill = fbuf[...][0]

    _phase_b(h1, fill, psrc_f, pdl_f, rows_v, acc, None, gsem)

    pltpu.sync_copy(acc.at[pl.ds(0, OWN)], agg_out.at[pl.ds(lo, OWN)])


def _make_k2():
    return pl.kernel(
        _k2_body,
        out_type=jax.ShapeDtypeStruct((NA, D), _f32),
        mesh=plsc.VectorSubcoreMesh(**_mesh),
        scratch_types=[
            pltpu.VMEM((CAP,), _i32),
            pltpu.VMEM((CAP,), _i32),
            pltpu.VMEM((ROW, D), _f32),
            pltpu.VMEM((ACC, D), _f32),
            pltpu.VMEM((16,), _i32),
            pltpu.SemaphoreType.DMA,
        ],
        **_params,
    )


def _k3_body(h, srcp, dstp,
             hs_out, hd_out,
             sidx, didx, hs_v, hd_v, gs, gd, ws, wd):
    c = lax.axis_index("c")
    s = lax.axis_index("s")
    w = s * NC + c

    pltpu.sync_copy(srcp.at[pl.ds(w * RPW, RPW)], sidx)
    pltpu.sync_copy(dstp.at[pl.ds(w * RPW, RPW)], didx)

    def gather(t, p):
        pltpu.async_copy(h.at[sidx.at[t]], hs_v.at[p], gs)
        pltpu.async_copy(h.at[didx.at[t]], hd_v.at[p], gd)

    def gwait(t, p):
        pltpu.make_async_copy(h.at[sidx.at[t]], hs_v.at[p], gs).wait()
        pltpu.make_async_copy(h.at[didx.at[t]], hd_v.at[p], gd).wait()

    def write(r, p):
        pltpu.async_copy(hs_v.at[p], hs_out.at[pl.ds(r * ROW, ROW)], ws)
        pltpu.async_copy(hd_v.at[p], hd_out.at[pl.ds(r * ROW, ROW)], wd)

    def wwait(r, p):
        pltpu.make_async_copy(hs_v.at[p], hs_out.at[pl.ds(r * ROW, ROW)],
                              ws).wait()
        pltpu.make_async_copy(hd_v.at[p], hd_out.at[pl.ds(r * ROW, ROW)],
                              wd).wait()

    gather(0, 0)

    def row_body(t, carry):
        r = w * RPW + t
        p = t % 2

        @pl.when(r < NROWS)
        def _():
            @pl.when((t + 1 < RPW) & (r + 1 < NROWS))
            def _():
                @pl.when(t >= 1)
                def _():
                    wwait(r - 1, 1 - p)

                gather(t + 1, 1 - p)

            gwait(t, p)
            write(r, p)

        return carry

    lax.fori_loop(0, RPW, row_body, 0)
    last = jnp.minimum(w * RPW + RPW, NROWS) - 1 - w * RPW

    @pl.when(last >= 0)
    def _():
        @pl.when(last >= 1)
        def _():
            wwait(w * RPW + last - 1, (last - 1) % 2)

        wwait(w * RPW + last, last % 2)


def _make_k3():
    return pl.kernel(
        _k3_body,
        out_type=(
            jax.ShapeDtypeStruct((NROWS_PAD * ROW, D), _f32),
            jax.ShapeDtypeStruct((NROWS_PAD * ROW, D), _f32),
        ),
        mesh=plsc.VectorSubcoreMesh(**_mesh),
        scratch_types=[
            pltpu.VMEM((RPW, ROW), _i32),
            pltpu.VMEM((RPW, ROW), _i32),
            pltpu.VMEM((2, ROW, D), _f32),
            pltpu.VMEM((2, ROW, D), _f32),
            pltpu.SemaphoreType.DMA,
            pltpu.SemaphoreType.DMA,
            pltpu.SemaphoreType.DMA,
            pltpu.SemaphoreType.DMA,
        ],
        **_params,
    )


def _dot_tc_body(hs, hd, o):
    o[...] = jnp.sum(hs[...] * hd[...], axis=-1)


def _dot_tc(hs3, hd3):
    blk = 128
    return pl.pallas_call(
        _dot_tc_body,
        grid=(NROWS_PAD // blk,),
        in_specs=[
            pl.BlockSpec((blk, ROW, D), lambda i: (i, 0, 0)),
            pl.BlockSpec((blk, ROW, D), lambda i: (i, 0, 0)),
        ],
        out_specs=pl.BlockSpec((blk, ROW), lambda i: (i, 0)),
        out_shape=jax.ShapeDtypeStruct((NROWS_PAD, ROW), _f32),
    )(hs3, hd3)


def _sage_tc_body(relu, agg, cnt, xin, wl, b, wr, o):
    c1 = jnp.maximum(cnt[:, 0:1], 1.0)
    mean = agg[...] / c1
    h = (jnp.dot(mean, wl[...], preferred_element_type=_f32,
                 precision=lax.Precision.HIGHEST)
         + b[...]
         + jnp.dot(xin[...], wr[...], preferred_element_type=_f32,
                   precision=lax.Precision.HIGHEST))
    o[...] = jnp.maximum(h, 0.0) if relu else h


def _sage_tc(relu, agg, cnt, xin, wlT, b, wrT):
    blk = 1000
    return pl.pallas_call(
        functools.partial(_sage_tc_body, relu),
        grid=(N // blk,),
        in_specs=[
            pl.BlockSpec((blk, D), lambda i: (i, 0)),
            pl.BlockSpec((blk, 16), lambda i: (i, 0)),
            pl.BlockSpec((blk, D), lambda i: (i, 0)),
            pl.BlockSpec((D, D), lambda i: (0, 0)),
            pl.BlockSpec((1, D), lambda i: (0, 0)),
            pl.BlockSpec((D, D), lambda i: (0, 0)),
        ],
        out_specs=pl.BlockSpec((blk, D), lambda i: (i, 0)),
        out_shape=jax.ShapeDtypeStruct((N, D), _f32),
    )(agg, cnt, xin, wlT, b, wrT)


def kernel(n_id, edge_label_index, emb, W1l, b1l, W1r, W2l, b2l, W2r):
    pad_rows = NROWS_PAD - NROWS
    src2 = jnp.concatenate(
        [edge_label_index[0], jnp.zeros((pad_rows * ROW,), _i32)]).reshape(
        NROWS_PAD, ROW)
    dst2 = jnp.concatenate(
        [edge_label_index[-1],
         jnp.full((pad_rows * ROW,), NA, _i32)]).reshape(NROWS_PAD, ROW)
    nidp = jnp.concatenate([n_id, jnp.zeros((N_PAD - N,), _i32)])
    z128 = jnp.zeros((NA, D), _f32)
    z16 = jnp.zeros((NA, 16), _f32)
    b1 = b1l.reshape(1, D)
    b2 = b2l.reshape(1, D)

    x_pad = _make_k0()(emb, nidp)
    agg1, cnt, psrc, pdl, fills = _make_k1()(
        x_pad, src2, dst2, z128, z16)
    h1 = _sage_tc(True, agg1[:N], cnt[:N], x_pad[:N], W1l.T, b1, W1r.T)
    agg2 = _make_k2()(h1, psrc, pdl, fills, z128)
    h = _sage_tc(False, agg2[:N], cnt[:N], h1, W2l.T, b2, W2r.T)
    hs, hd = _make_k3()(h, src2, dst2)
    pred = _dot_tc(hs.reshape(NROWS_PAD, ROW, D), hd.reshape(NROWS_PAD, ROW, D))
    return pred.reshape(NROWS_PAD * ROW)[:E]


# speedup vs baseline: 3.4110x; 1.0547x over previous
import functools

import jax
import jax.numpy as jnp
from jax import lax
from jax.experimental import pallas as pl
from jax.experimental.pallas import tpu as pltpu
from jax.experimental.pallas import tpu_sc as plsc

N = 10000
E = 320000
D = 128
NC = 2
NS = 16
NW = NC * NS
ROW = 128
NROWS = E // ROW
RPW = 80
NROWS_PAD = RPW * NW
NA = 10240
OWN = NA // NW
ACC = OWN + 8
CAP = 11136
NB = CAP // ROW
SCH = 32
N_PAD = 10240
XB = N_PAD // 128

_f32 = jnp.float32
_i32 = jnp.int32
_mesh = dict(core_axis_name="c", subcore_axis_name="s")
_params = dict(compiler_params=pltpu.CompilerParams(needs_layout_passes=False))


def _k0_body(emb, nidp, x_out, nid_v, rows_v, gsem):
    c = lax.axis_index("c")
    s = lax.axis_index("s")
    w = s * NC + c
    pltpu.sync_copy(nidp, nid_v)

    def xb_body(t, carry):
        b = w + NW * t

        @pl.when(b < XB)
        def _():
            pltpu.async_copy(emb.at[nid_v.at[pl.ds(b * 128, 128)]],
                             rows_v, gsem).wait()
            pltpu.sync_copy(rows_v, x_out.at[pl.ds(b * 128, 128)])

        return carry

    lax.fori_loop(0, -(-XB // NW), xb_body, 0)


def _make_k0():
    return pl.kernel(
        _k0_body,
        out_type=jax.ShapeDtypeStruct((N_PAD, D), _f32),
        mesh=plsc.VectorSubcoreMesh(**_mesh),
        scratch_types=[
            pltpu.VMEM((N_PAD,), _i32),
            pltpu.VMEM((ROW, D), _f32),
            pltpu.SemaphoreType.DMA,
        ],
        **_params,
    )


def _accum(pend_f, rows, acc, cntacc, b):
    ones16 = jnp.ones((16,), _f32)

    def g_body(g, carry2):
        dl16 = pend_f[pl.ds(b * ROW + g * 16, 16)] & 1023
        for l in range(16):
            dle = dl16[l]
            e = g * 16 + l
            for k in range(8):
                plsc.addupdate(acc.at[dle, pl.ds(k * 16, 16)],
                               rows[e, pl.ds(k * 16, 16)])
            if cntacc is not None:
                plsc.addupdate(cntacc.at[dle, pl.ds(0, 16)], ones16)
        return carry2

    lax.fori_loop(0, 8, g_body, 0)


def _unpack_src(pend_f, sstage, b):
    for g in range(8):
        v16 = pend_f[pl.ds(b * ROW + g * 16, 16)]
        sstage[pl.ds(g * 16, 16)] = lax.shift_right_logical(v16, 10)


def _phase_b1(table, fill, pend_f, sstage, rows_v, acc, cntacc, gsem):
    def b_body(b, carry):
        @pl.when(b * ROW < fill)
        def _():
            _unpack_src(pend_f, sstage, b)
            pltpu.async_copy(table.at[sstage], rows_v, gsem).wait()
            _accum(pend_f, rows_v, acc, cntacc, b)

        return carry

    lax.fori_loop(0, NB, b_body, 0)


def _phase_b2(table, fill, pend_f, sstage2, rows_v2, acc, gsem):
    def prep_issue(b, p):
        _unpack_src(pend_f, sstage2.at[p], b)
        pltpu.async_copy(table.at[sstage2.at[p]], rows_v2.at[p], gsem)

    @pl.when(0 < fill)
    def _():
        prep_issue(0, 0)

    def b_body(b, carry):
        p = b % 2

        @pl.when(b * ROW < fill)
        def _():
            @pl.when((b + 1) * ROW < fill)
            def _():
                prep_issue(b + 1, 1 - p)

            pltpu.make_async_copy(table.at[sstage2.at[p]],
                                  rows_v2.at[p], gsem).wait()
            _accum(pend_f, rows_v2.at[p], acc, None, b)

        return carry

    lax.fori_loop(0, NB, b_body, 0)


def _k1_body(x, srcp, dstp, z128, z16,
             agg_out, cnt_out, pend_out, fill_out,
             sidx, didx, pend_f, sstage, rows_v,
             acc, cntacc, fbuf, gsem):
    c = lax.axis_index("c")
    s = lax.axis_index("s")
    w = s * NC + c
    lo = w * OWN

    pltpu.sync_copy(z128.at[pl.ds(0, ACC)], acc)
    pltpu.sync_copy(z16.at[pl.ds(0, ACC)], cntacc)

    def chunk_body(q, fill):
        pltpu.sync_copy(srcp.at[pl.ds(q * SCH, SCH)], sidx)
        pltpu.sync_copy(dstp.at[pl.ds(q * SCH, SCH)], didx)

        def row_body(rr, fill2):
            f = fill2
            for j in range(8):
                d = didx[rr, pl.ds(j * 16, 16)]
                sv = sidx[rr, pl.ds(j * 16, 16)]
                m = (d >= lo) & (d < lo + OWN)
                plsc.store_compressed(pend_f.at[pl.ds(f, 16)],
                                      sv * 1024 + (d - lo), mask=m)
                nv = plsc.all_reduce_population_count(m)
                f = f + nv[0]
            return f

        return lax.fori_loop(0, SCH, row_body, fill)

    fill = lax.fori_loop(0, NROWS_PAD // SCH, chunk_body, 0)

    pad_v = jnp.full((16,), OWN, _i32)
    for k in range(8):
        pend_f[pl.ds(fill + k * 16, 16)] = pad_v

    pltpu.sync_copy(pend_f, pend_out.at[w])
    fbuf[...] = jnp.zeros((16,), _i32) + fill
    pltpu.sync_copy(fbuf, fill_out.at[w])

    _phase_b1(x, fill, pend_f, sstage, rows_v, acc, cntacc, gsem)

    pltpu.sync_copy(acc.at[pl.ds(0, OWN)], agg_out.at[pl.ds(lo, OWN)])
    pltpu.sync_copy(cntacc.at[pl.ds(0, OWN)], cnt_out.at[pl.ds(lo, OWN)])


def _make_k1():
    return pl.kernel(
        _k1_body,
        out_type=(
            jax.ShapeDtypeStruct((NA, D), _f32),
            jax.ShapeDtypeStruct((NA, 16), _f32),
            jax.ShapeDtypeStruct((NW, CAP), _i32),
            jax.ShapeDtypeStruct((NW, 16), _i32),
        ),
        mesh=plsc.VectorSubcoreMesh(**_mesh),
        scratch_types=[
            pltpu.VMEM((SCH, ROW), _i32),
            pltpu.VMEM((SCH, ROW), _i32),
            pltpu.VMEM((CAP,), _i32),
            pltpu.VMEM((ROW,), _i32),
            pltpu.VMEM((ROW, D), _f32),
            pltpu.VMEM((ACC, D), _f32),
            pltpu.VMEM((ACC, 16), _f32),
            pltpu.VMEM((16,), _i32),
            pltpu.SemaphoreType.DMA,
        ],
        **_params,
    )


def _k2_body(h1, pend_in, fill_in, z128,
             agg_out,
             pend_f, sstage2, rows_v2, acc, fbuf, gsem):
    c = lax.axis_index("c")
    s = lax.axis_index("s")
    w = s * NC + c
    lo = w * OWN

    pltpu.sync_copy(z128.at[pl.ds(0, ACC)], acc)
    pltpu.sync_copy(pend_in.at[w], pend_f)
    pltpu.sync_copy(fill_in.at[w], fbuf)
    fill = fbuf[...][0]

    _phase_b2(h1, fill, pend_f, sstage2, rows_v2, acc, gsem)

    pltpu.sync_copy(acc.at[pl.ds(0, OWN)], agg_out.at[pl.ds(lo, OWN)])


def _make_k2():
    return pl.kernel(
        _k2_body,
        out_type=jax.ShapeDtypeStruct((NA, D), _f32),
        mesh=plsc.VectorSubcoreMesh(**_mesh),
        scratch_types=[
            pltpu.VMEM((CAP,), _i32),
            pltpu.VMEM((2, ROW), _i32),
            pltpu.VMEM((2, ROW, D), _f32),
            pltpu.VMEM((ACC, D), _f32),
            pltpu.VMEM((16,), _i32),
            pltpu.SemaphoreType.DMA,
        ],
        **_params,
    )


def _k3_body(h, srcp, dstp,
             hs_out, hd_out,
             sidx, didx, hs_v, hd_v, gs, gd, ws, wd):
    c = lax.axis_index("c")
    s = lax.axis_index("s")
    w = s * NC + c

    pltpu.sync_copy(srcp.at[pl.ds(w * RPW, RPW)], sidx)
    pltpu.sync_copy(dstp.at[pl.ds(w * RPW, RPW)], didx)

    def gather(t, p):
        pltpu.async_copy(h.at[sidx.at[t]], hs_v.at[p], gs)
        pltpu.async_copy(h.at[didx.at[t]], hd_v.at[p], gd)

    def gwait(t, p):
        pltpu.make_async_copy(h.at[sidx.at[t]], hs_v.at[p], gs).wait()
        pltpu.make_async_copy(h.at[didx.at[t]], hd_v.at[p], gd).wait()

    def write(r, p):
        pltpu.async_copy(hs_v.at[p], hs_out.at[pl.ds(r * ROW, ROW)], ws)
        pltpu.async_copy(hd_v.at[p], hd_out.at[pl.ds(r * ROW, ROW)], wd)

    def wwait(r, p):
        pltpu.make_async_copy(hs_v.at[p], hs_out.at[pl.ds(r * ROW, ROW)],
                              ws).wait()
        pltpu.make_async_copy(hd_v.at[p], hd_out.at[pl.ds(r * ROW, ROW)],
                              wd).wait()

    gather(0, 0)

    def row_body(t, carry):
        r = w * RPW + t
        p = t % 2

        @pl.when(r < NROWS)
        def _():
            @pl.when((t + 1 < RPW) & (r + 1 < NROWS))
            def _():
                @pl.when(t >= 1)
                def _():
                    wwait(r - 1, 1 - p)

                gather(t + 1, 1 - p)

            gwait(t, p)
            write(r, p)

        return carry

    lax.fori_loop(0, RPW, row_body, 0)
    last = jnp.minimum(w * RPW + RPW, NROWS) - 1 - w * RPW

    @pl.when(last >= 0)
    def _():
        @pl.when(last >= 1)
        def _():
            wwait(w * RPW + last - 1, (last - 1) % 2)

        wwait(w * RPW + last, last % 2)


def _make_k3():
    return pl.kernel(
        _k3_body,
        out_type=(
            jax.ShapeDtypeStruct((NROWS_PAD * ROW, D), _f32),
            jax.ShapeDtypeStruct((NROWS_PAD * ROW, D), _f32),
        ),
        mesh=plsc.VectorSubcoreMesh(**_mesh),
        scratch_types=[
            pltpu.VMEM((RPW, ROW), _i32),
            pltpu.VMEM((RPW, ROW), _i32),
            pltpu.VMEM((2, ROW, D), _f32),
            pltpu.VMEM((2, ROW, D), _f32),
            pltpu.SemaphoreType.DMA,
            pltpu.SemaphoreType.DMA,
            pltpu.SemaphoreType.DMA,
            pltpu.SemaphoreType.DMA,
        ],
        **_params,
    )


def _dot_tc_body(hs, hd, o):
    o[...] = jnp.sum(hs[...] * hd[...], axis=-1)


def _dot_tc(hs3, hd3):
    blk = 128
    return pl.pallas_call(
        _dot_tc_body,
        grid=(NROWS_PAD // blk,),
        in_specs=[
            pl.BlockSpec((blk, ROW, D), lambda i: (i, 0, 0)),
            pl.BlockSpec((blk, ROW, D), lambda i: (i, 0, 0)),
        ],
        out_specs=pl.BlockSpec((blk, ROW), lambda i: (i, 0)),
        out_shape=jax.ShapeDtypeStruct((NROWS_PAD, ROW), _f32),
    )(hs3, hd3)


def _sage_tc_body(relu, agg, cnt, xin, wl, b, wr, o):
    c1 = jnp.maximum(cnt[:, 0:1], 1.0)
    mean = agg[...] / c1
    h = (jnp.dot(mean, wl[...], preferred_element_type=_f32,
                 precision=lax.Precision.HIGHEST)
         + b[...]
         + jnp.dot(xin[...], wr[...], preferred_element_type=_f32,
                   precision=lax.Precision.HIGHEST))
    o[...] = jnp.maximum(h, 0.0) if relu else h


def _sage_tc(relu, agg, cnt, xin, wlT, b, wrT):
    blk = 1000
    return pl.pallas_call(
        functools.partial(_sage_tc_body, relu),
        grid=(N // blk,),
        in_specs=[
            pl.BlockSpec((blk, D), lambda i: (i, 0)),
            pl.BlockSpec((blk, 16), lambda i: (i, 0)),
            pl.BlockSpec((blk, D), lambda i: (i, 0)),
            pl.BlockSpec((D, D), lambda i: (0, 0)),
            pl.BlockSpec((1, D), lambda i: (0, 0)),
            pl.BlockSpec((D, D), lambda i: (0, 0)),
        ],
        out_specs=pl.BlockSpec((blk, D), lambda i: (i, 0)),
        out_shape=jax.ShapeDtypeStruct((N, D), _f32),
    )(agg, cnt, xin, wlT, b, wrT)


def kernel(n_id, edge_label_index, emb, W1l, b1l, W1r, W2l, b2l, W2r):
    pad_rows = NROWS_PAD - NROWS
    src2 = jnp.concatenate(
        [edge_label_index[0], jnp.zeros((pad_rows * ROW,), _i32)]).reshape(
        NROWS_PAD, ROW)
    dst2 = jnp.concatenate(
        [edge_label_index[-1],
         jnp.full((pad_rows * ROW,), NA, _i32)]).reshape(NROWS_PAD, ROW)
    nidp = jnp.concatenate([n_id, jnp.zeros((N_PAD - N,), _i32)])
    z128 = jnp.zeros((NA, D), _f32)
    z16 = jnp.zeros((NA, 16), _f32)
    b1 = b1l.reshape(1, D)
    b2 = b2l.reshape(1, D)

    x_pad = _make_k0()(emb, nidp)
    agg1, cnt, pend, fills = _make_k1()(x_pad, src2, dst2, z128, z16)
    h1 = _sage_tc(True, agg1[:N], cnt[:N], x_pad[:N], W1l.T, b1, W1r.T)
    agg2 = _make_k2()(h1, pend, fills, z128)
    h = _sage_tc(False, agg2[:N], cnt[:N], h1, W2l.T, b2, W2r.T)
    hs, hd = _make_k3()(h, src2, dst2)
    pred = _dot_tc(hs.reshape(NROWS_PAD, ROW, D), hd.reshape(NROWS_PAD, ROW, D))
    return pred.reshape(NROWS_PAD * ROW)[:E]


# speedup vs baseline: 3.5517x; 1.0413x over previous
import functools

import jax
import jax.numpy as jnp
from jax import lax
from jax.experimental import pallas as pl
from jax.experimental.pallas import tpu as pltpu
from jax.experimental.pallas import tpu_sc as plsc

N = 10000
E = 320000
D = 128
NC = 2
NS = 16
NW = NC * NS
ROW = 128
NROWS = E // ROW
RPW = 80
NROWS_PAD = RPW * NW
NA = 10240
OWN = NA // NW
ACC = OWN + 8
CAP = 11136
NB = CAP // ROW
SCH = 32
N_PAD = 10240
XB = N_PAD // 128

_f32 = jnp.float32
_i32 = jnp.int32
_mesh = dict(core_axis_name="c", subcore_axis_name="s")
_params = dict(compiler_params=pltpu.CompilerParams(needs_layout_passes=False))


def _k0_body(emb, nidp, x_out, nid_v, rows_v, gsem):
    c = lax.axis_index("c")
    s = lax.axis_index("s")
    w = s * NC + c
    pltpu.sync_copy(nidp, nid_v)

    def xb_body(t, carry):
        b = w + NW * t

        @pl.when(b < XB)
        def _():
            pltpu.async_copy(emb.at[nid_v.at[pl.ds(b * 128, 128)]],
                             rows_v, gsem).wait()
            pltpu.sync_copy(rows_v, x_out.at[pl.ds(b * 128, 128)])

        return carry

    lax.fori_loop(0, -(-XB // NW), xb_body, 0)


def _make_k0():
    return pl.kernel(
        _k0_body,
        out_type=jax.ShapeDtypeStruct((N_PAD, D), _f32),
        mesh=plsc.VectorSubcoreMesh(**_mesh),
        scratch_types=[
            pltpu.VMEM((N_PAD,), _i32),
            pltpu.VMEM((ROW, D), _f32),
            pltpu.SemaphoreType.DMA,
        ],
        **_params,
    )


def _accum(pend_f, rows, acc, cntacc, b):
    ones16 = jnp.ones((16,), _f32)

    def g_body(g, carry2):
        dl16 = pend_f[pl.ds(b * ROW + g * 16, 16)] & 1023
        for l in range(16):
            dle = dl16[l]
            e = g * 16 + l
            for k in range(8):
                plsc.addupdate(acc.at[dle, pl.ds(k * 16, 16)],
                               rows[e, pl.ds(k * 16, 16)])
            if cntacc is not None:
                plsc.addupdate(cntacc.at[dle, pl.ds(0, 16)], ones16)
        return carry2

    lax.fori_loop(0, 8, g_body, 0)


def _unpack_src(pend_f, sstage, b):
    for g in range(8):
        v16 = pend_f[pl.ds(b * ROW + g * 16, 16)]
        sstage[pl.ds(g * 16, 16)] = lax.shift_right_logical(v16, 10)


def _phase_b1(table, fill, pend_f, sstage, rows_v, acc, cntacc, gsem):
    def b_body(b, carry):
        @pl.when(b * ROW < fill)
        def _():
            _unpack_src(pend_f, sstage, b)
            pltpu.async_copy(table.at[sstage], rows_v, gsem).wait()
            _accum(pend_f, rows_v, acc, cntacc, b)

        return carry

    lax.fori_loop(0, NB, b_body, 0)


def _phase_b2(table, fill, pend_f, sstage2, rows_v2, acc, cntacc, gsem):
    def prep_issue(b, p):
        _unpack_src(pend_f, sstage2.at[p], b)
        pltpu.async_copy(table.at[sstage2.at[p]], rows_v2.at[p], gsem)

    @pl.when(0 < fill)
    def _():
        prep_issue(0, 0)

    def b_body(b, carry):
        p = b % 2

        @pl.when(b * ROW < fill)
        def _():
            @pl.when((b + 1) * ROW < fill)
            def _():
                prep_issue(b + 1, 1 - p)

            pltpu.make_async_copy(table.at[sstage2.at[p]],
                                  rows_v2.at[p], gsem).wait()
            _accum(pend_f, rows_v2.at[p], acc, cntacc, b)

        return carry

    lax.fori_loop(0, NB, b_body, 0)


def _kscan_body(srcp, dstp,
                pend_out, fill_out,
                sidx, didx, pend_f, fbuf):
    c = lax.axis_index("c")
    s = lax.axis_index("s")
    w = s * NC + c
    lo = w * OWN

    def chunk_body(q, fill):
        pltpu.sync_copy(srcp.at[pl.ds(q * SCH, SCH)], sidx)
        pltpu.sync_copy(dstp.at[pl.ds(q * SCH, SCH)], didx)

        def row_body(rr, fill2):
            f = fill2
            for j in range(8):
                d = didx[rr, pl.ds(j * 16, 16)]
                sv = sidx[rr, pl.ds(j * 16, 16)]
                m = (d >= lo) & (d < lo + OWN)
                plsc.store_compressed(pend_f.at[pl.ds(f, 16)],
                                      sv * 1024 + (d - lo), mask=m)
                nv = plsc.all_reduce_population_count(m)
                f = f + nv[0]
            return f

        return lax.fori_loop(0, SCH, row_body, fill)

    fill = lax.fori_loop(0, NROWS_PAD // SCH, chunk_body, 0)

    pad_v = jnp.full((16,), OWN, _i32)
    for k in range(8):
        pend_f[pl.ds(fill + k * 16, 16)] = pad_v

    pltpu.sync_copy(pend_f, pend_out.at[w])
    fbuf[...] = jnp.zeros((16,), _i32) + fill
    pltpu.sync_copy(fbuf, fill_out.at[w])


def _make_kscan():
    return pl.kernel(
        _kscan_body,
        out_type=(
            jax.ShapeDtypeStruct((NW, CAP), _i32),
            jax.ShapeDtypeStruct((NW, 16), _i32),
        ),
        mesh=plsc.VectorSubcoreMesh(**_mesh),
        scratch_types=[
            pltpu.VMEM((SCH, ROW), _i32),
            pltpu.VMEM((SCH, ROW), _i32),
            pltpu.VMEM((CAP,), _i32),
            pltpu.VMEM((16,), _i32),
        ],
        **_params,
    )


def _kagg_body(with_cnt, *args):
    if with_cnt:
        (table, pend_in, fill_in, z128, z16,
         agg_out, cnt_out,
         pend_f, sstage2, rows_v2, acc, cntacc, fbuf, gsem) = args
    else:
        (table, pend_in, fill_in, z128,
         agg_out,
         pend_f, sstage2, rows_v2, acc, fbuf, gsem) = args
        cntacc = None
    c = lax.axis_index("c")
    s = lax.axis_index("s")
    w = s * NC + c
    lo = w * OWN

    pltpu.sync_copy(z128.at[pl.ds(0, ACC)], acc)
    if with_cnt:
        pltpu.sync_copy(z16.at[pl.ds(0, ACC)], cntacc)
    pltpu.sync_copy(pend_in.at[w], pend_f)
    pltpu.sync_copy(fill_in.at[w], fbuf)
    fill = fbuf[...][0]

    _phase_b2(table, fill, pend_f, sstage2, rows_v2, acc, cntacc, gsem)

    pltpu.sync_copy(acc.at[pl.ds(0, OWN)], agg_out.at[pl.ds(lo, OWN)])
    if with_cnt:
        pltpu.sync_copy(cntacc.at[pl.ds(0, OWN)], cnt_out.at[pl.ds(lo, OWN)])


def _make_kagg(with_cnt):
    out = [jax.ShapeDtypeStruct((NA, D), _f32)]
    scratch = [
        pltpu.VMEM((CAP,), _i32),
        pltpu.VMEM((2, ROW), _i32),
        pltpu.VMEM((2, ROW, D), _f32),
        pltpu.VMEM((ACC, D), _f32),
    ]
    if with_cnt:
        out.append(jax.ShapeDtypeStruct((NA, 16), _f32))
        scratch.append(pltpu.VMEM((ACC, 16), _f32))
    scratch += [pltpu.VMEM((16,), _i32), pltpu.SemaphoreType.DMA]
    return pl.kernel(
        functools.partial(_kagg_body, with_cnt),
        out_type=tuple(out),
        mesh=plsc.VectorSubcoreMesh(**_mesh),
        scratch_types=scratch,
        **_params,
    )


def _k3_body(h, srcp, dstp,
             hs_out, hd_out,
             sidx, didx, hs_v, hd_v, gs, gd, ws, wd):
    c = lax.axis_index("c")
    s = lax.axis_index("s")
    w = s * NC + c

    pltpu.sync_copy(srcp.at[pl.ds(w * RPW, RPW)], sidx)
    pltpu.sync_copy(dstp.at[pl.ds(w * RPW, RPW)], didx)

    def gather(t, p):
        pltpu.async_copy(h.at[sidx.at[t]], hs_v.at[p], gs)
        pltpu.async_copy(h.at[didx.at[t]], hd_v.at[p], gd)

    def gwait(t, p):
        pltpu.make_async_copy(h.at[sidx.at[t]], hs_v.at[p], gs).wait()
        pltpu.make_async_copy(h.at[didx.at[t]], hd_v.at[p], gd).wait()

    def write(r, p):
        pltpu.async_copy(hs_v.at[p], hs_out.at[pl.ds(r * ROW, ROW)], ws)
        pltpu.async_copy(hd_v.at[p], hd_out.at[pl.ds(r * ROW, ROW)], wd)

    def wwait(r, p):
        pltpu.make_async_copy(hs_v.at[p], hs_out.at[pl.ds(r * ROW, ROW)],
                              ws).wait()
        pltpu.make_async_copy(hd_v.at[p], hd_out.at[pl.ds(r * ROW, ROW)],
                              wd).wait()

    gather(0, 0)

    def row_body(t, carry):
        r = w * RPW + t
        p = t % 2

        @pl.when(r < NROWS)
        def _():
            @pl.when((t + 1 < RPW) & (r + 1 < NROWS))
            def _():
                @pl.when(t >= 1)
                def _():
                    wwait(r - 1, 1 - p)

                gather(t + 1, 1 - p)

            gwait(t, p)
            write(r, p)

        return carry

    lax.fori_loop(0, RPW, row_body, 0)
    last = jnp.minimum(w * RPW + RPW, NROWS) - 1 - w * RPW

    @pl.when(last >= 0)
    def _():
        @pl.when(last >= 1)
        def _():
            wwait(w * RPW + last - 1, (last - 1) % 2)

        wwait(w * RPW + last, last % 2)


def _make_k3():
    return pl.kernel(
        _k3_body,
        out_type=(
            jax.ShapeDtypeStruct((NROWS_PAD * ROW, D), _f32),
            jax.ShapeDtypeStruct((NROWS_PAD * ROW, D), _f32),
        ),
        mesh=plsc.VectorSubcoreMesh(**_mesh),
        scratch_types=[
            pltpu.VMEM((RPW, ROW), _i32),
            pltpu.VMEM((RPW, ROW), _i32),
            pltpu.VMEM((2, ROW, D), _f32),
            pltpu.VMEM((2, ROW, D), _f32),
            pltpu.SemaphoreType.DMA,
            pltpu.SemaphoreType.DMA,
            pltpu.SemaphoreType.DMA,
            pltpu.SemaphoreType.DMA,
        ],
        **_params,
    )


def _dot_tc_body(hs, hd, o):
    o[...] = jnp.sum(hs[...] * hd[...], axis=-1)


def _dot_tc(hs3, hd3):
    blk = 128
    return pl.pallas_call(
        _dot_tc_body,
        grid=(NROWS_PAD // blk,),
        in_specs=[
            pl.BlockSpec((blk, ROW, D), lambda i: (i, 0, 0)),
            pl.BlockSpec((blk, ROW, D), lambda i: (i, 0, 0)),
        ],
        out_specs=pl.BlockSpec((blk, ROW), lambda i: (i, 0)),
        out_shape=jax.ShapeDtypeStruct((NROWS_PAD, ROW), _f32),
    )(hs3, hd3)


def _sage_tc_body(relu, agg, cnt, xin, wl, b, wr, o):
    c1 = jnp.maximum(cnt[:, 0:1], 1.0)
    mean = agg[...] / c1
    h = (jnp.dot(mean, wl[...], preferred_element_type=_f32,
                 precision=lax.Precision.HIGHEST)
         + b[...]
         + jnp.dot(xin[...], wr[...], preferred_element_type=_f32,
                   precision=lax.Precision.HIGHEST))
    o[...] = jnp.maximum(h, 0.0) if relu else h


def _sage_tc(relu, agg, cnt, xin, wlT, b, wrT):
    blk = 1000
    return pl.pallas_call(
        functools.partial(_sage_tc_body, relu),
        grid=(N // blk,),
        in_specs=[
            pl.BlockSpec((blk, D), lambda i: (i, 0)),
            pl.BlockSpec((blk, 16), lambda i: (i, 0)),
            pl.BlockSpec((blk, D), lambda i: (i, 0)),
            pl.BlockSpec((D, D), lambda i: (0, 0)),
            pl.BlockSpec((1, D), lambda i: (0, 0)),
            pl.BlockSpec((D, D), lambda i: (0, 0)),
        ],
        out_specs=pl.BlockSpec((blk, D), lambda i: (i, 0)),
        out_shape=jax.ShapeDtypeStruct((N, D), _f32),
    )(agg, cnt, xin, wlT, b, wrT)


def kernel(n_id, edge_label_index, emb, W1l, b1l, W1r, W2l, b2l, W2r):
    pad_rows = NROWS_PAD - NROWS
    src2 = jnp.concatenate(
        [edge_label_index[0], jnp.zeros((pad_rows * ROW,), _i32)]).reshape(
        NROWS_PAD, ROW)
    dst2 = jnp.concatenate(
        [edge_label_index[-1],
         jnp.full((pad_rows * ROW,), NA, _i32)]).reshape(NROWS_PAD, ROW)
    nidp = jnp.concatenate([n_id, jnp.zeros((N_PAD - N,), _i32)])
    z128 = jnp.zeros((NA, D), _f32)
    z16 = jnp.zeros((NA, 16), _f32)
    b1 = b1l.reshape(1, D)
    b2 = b2l.reshape(1, D)

    x_pad = _make_k0()(emb, nidp)
    pend, fills = _make_kscan()(src2, dst2)
    agg1, cnt = _make_kagg(True)(x_pad, pend, fills, z128, z16)
    h1 = _sage_tc(True, agg1[:N], cnt[:N], x_pad[:N], W1l.T, b1, W1r.T)
    (agg2,) = _make_kagg(False)(h1, pend, fills, z128)
    h = _sage_tc(False, agg2[:N], cnt[:N], h1, W2l.T, b2, W2r.T)
    hs, hd = _make_k3()(h, src2, dst2)
    pred = _dot_tc(hs.reshape(NROWS_PAD, ROW, D), hd.reshape(NROWS_PAD, ROW, D))
    return pred.reshape(NROWS_PAD * ROW)[:E]
